# baseline (speedup 1.0000x reference)
import functools

import jax
import jax.numpy as jnp
from jax import lax
from jax.experimental import pallas as pl
from jax.experimental.pallas import tpu as pltpu
from jax.experimental.pallas import tpu_sc as plsc

NUM_CUSTOMERS = 1000000
NUM_ARTICLES = 100000
EMBED_DIM = 32
BATCH = 16384

NC, NS, L = 2, 16, 16
NW = NC * NS
BPW = BATCH // NW
CHUNK = 128
NCHUNK = BPW // CHUNK
NGROUP = BPW // L


def _sc_body(user_hbm, article_hbm, cust_hbm, art_hbm, out_hbm,
             idx_u, idx_a, u_rows, a_rows, out_v, sem_u, sem_a):
    wid = lax.axis_index("s") * NC + lax.axis_index("c")
    base = wid * BPW

    pltpu.sync_copy(user_hbm.at[pl.ds(wid * NCHUNK, NCHUNK)], idx_u)
    pltpu.sync_copy(article_hbm.at[pl.ds(wid * NCHUNK, NCHUNK)], idx_a)

    copies = []
    for j in range(NCHUNK):
        copies.append(pltpu.async_copy(
            cust_hbm.at[idx_u.at[j]], u_rows.at[pl.ds(j * CHUNK, CHUNK)], sem_u))
        copies.append(pltpu.async_copy(
            art_hbm.at[idx_a.at[j]], a_rows.at[pl.ds(j * CHUNK, CHUNK)], sem_a))
    for c in copies:
        c.wait()

    cols = [jnp.full((L,), d, jnp.int32) for d in range(EMBED_DIM)]

    def group(g, carry):
        rows = g * L + lax.iota(jnp.int32, L)
        acc = plsc.load_gather(u_rows, [rows, cols[0]]) * \
            plsc.load_gather(a_rows, [rows, cols[0]])
        for d in range(1, EMBED_DIM):
            acc = acc + plsc.load_gather(u_rows, [rows, cols[d]]) * \
                plsc.load_gather(a_rows, [rows, cols[d]])
        out_v[pl.ds(g * L, L)] = acc
        return carry

    lax.fori_loop(0, NGROUP, group, 0)
    pltpu.sync_copy(out_v, out_hbm.at[pl.ds(base, BPW)])


@jax.jit
def _recommend_sc(user, article, customer_table, article_table):
    mesh = plsc.VectorSubcoreMesh(core_axis_name="c", subcore_axis_name="s")
    kern = functools.partial(
        pl.kernel,
        mesh=mesh,
        out_type=jax.ShapeDtypeStruct((BATCH,), jnp.float32),
        scratch_types=[
            pltpu.VMEM((NCHUNK, CHUNK), jnp.int32),
            pltpu.VMEM((NCHUNK, CHUNK), jnp.int32),
            pltpu.VMEM((BPW, EMBED_DIM), jnp.float32),
            pltpu.VMEM((BPW, EMBED_DIM), jnp.float32),
            pltpu.VMEM((BPW,), jnp.float32),
            pltpu.SemaphoreType.DMA,
            pltpu.SemaphoreType.DMA,
        ],
        compiler_params=pltpu.CompilerParams(
            needs_layout_passes=False, use_tc_tiling_on_sc=False),
    )(_sc_body)
    return kern(user, article, customer_table, article_table)


def kernel(user, article, customer_table, article_table):
    user2d = user.reshape(BATCH // CHUNK, CHUNK)
    article2d = article.reshape(BATCH // CHUNK, CHUNK)
    scores = _recommend_sc(user2d, article2d, customer_table, article_table)
    return scores.reshape(BATCH, 1)
